# baseline (speedup 1.0000x reference)
import jax
import jax.numpy as jnp
from jax.experimental import pallas as pl


def _pos_body(xt_ref, yt_ref, out_ref):
    nf, w = xt_ref.shape
    h = yt_ref.shape[1]
    xt = xt_ref[...]
    yt = yt_ref[...]
    xcol = jnp.broadcast_to(xt[:, None, :], (nf, h, w))
    yrow = jnp.broadcast_to(yt[:, :, None], (nf, h, w))
    out_ref[0, :nf] = xcol
    out_ref[0, nf:] = yrow


def kernel(mask, feature_map, row_embed, col_embed):
    h, w = mask.shape[-2], mask.shape[-1]
    bs = mask.shape[0]
    nf = row_embed.shape[1]
    xt = col_embed[:w].T
    yt = row_embed[:h].T
    return pl.pallas_call(
        _pos_body,
        grid=(bs,),
        in_specs=[
            pl.BlockSpec((nf, w), lambda b: (0, 0)),
            pl.BlockSpec((nf, h), lambda b: (0, 0)),
        ],
        out_specs=pl.BlockSpec((1, 2 * nf, h, w), lambda b: (b, 0, 0, 0)),
        out_shape=jax.ShapeDtypeStruct((bs, 2 * nf, h, w), jnp.float32),
    )(xt, yt)


# speedup vs baseline: 1.0070x; 1.0070x over previous
import jax
import jax.numpy as jnp
from jax.experimental import pallas as pl
from jax.experimental.pallas import tpu as pltpu


def _pos_body(xt_ref, yt_ref, out_ref, scratch, sem):
    nf, w = xt_ref.shape
    h = yt_ref.shape[1]
    bs = out_ref.shape[0]
    xt = xt_ref[...]
    yt = yt_ref[...]
    scratch[:nf] = jnp.broadcast_to(xt[:, None, :], (nf, h, w))
    scratch[nf:] = jnp.broadcast_to(yt[:, :, None], (nf, h, w))
    copies = [
        pltpu.make_async_copy(scratch, out_ref.at[b], sem) for b in range(bs)
    ]
    for c in copies:
        c.start()
    for c in copies:
        c.wait()


def kernel(mask, feature_map, row_embed, col_embed):
    h, w = mask.shape[-2], mask.shape[-1]
    bs = mask.shape[0]
    nf = row_embed.shape[1]
    xt = col_embed[:w].T
    yt = row_embed[:h].T
    return pl.pallas_call(
        _pos_body,
        in_specs=[
            pl.BlockSpec(memory_space=pltpu.VMEM),
            pl.BlockSpec(memory_space=pltpu.VMEM),
        ],
        out_specs=pl.BlockSpec(memory_space=pl.ANY),
        out_shape=jax.ShapeDtypeStruct((bs, 2 * nf, h, w), jnp.float32),
        scratch_shapes=[
            pltpu.VMEM((2 * nf, h, w), jnp.float32),
            pltpu.SemaphoreType.DMA,
        ],
    )(xt, yt)


# speedup vs baseline: 11.3478x; 11.2685x over previous
import jax
import jax.numpy as jnp
from jax.experimental import pallas as pl
from jax.experimental.pallas import tpu as pltpu


def _pos_body(col_ref, row_ref, out_ref, scratch, sem):
    nf = col_ref.shape[1]
    h, w = scratch.shape[0], scratch.shape[1]
    bs = out_ref.shape[0]
    ce = col_ref[:w, :]
    re = row_ref[:h, :]
    scratch[:, :, :nf] = jnp.broadcast_to(ce[None, :, :], (h, w, nf))
    scratch[:, :, nf:] = jnp.broadcast_to(re[:, None, :], (h, w, nf))
    copies = [
        pltpu.make_async_copy(scratch, out_ref.at[b], sem) for b in range(bs)
    ]
    for c in copies:
        c.start()
    for c in copies:
        c.wait()


def kernel(mask, feature_map, row_embed, col_embed):
    h, w = mask.shape[-2], mask.shape[-1]
    bs = mask.shape[0]
    nf = row_embed.shape[1]
    q = pl.pallas_call(
        _pos_body,
        in_specs=[
            pl.BlockSpec(memory_space=pltpu.VMEM),
            pl.BlockSpec(memory_space=pltpu.VMEM),
        ],
        out_specs=pl.BlockSpec(memory_space=pl.ANY),
        out_shape=jax.ShapeDtypeStruct((bs, h, w, 2 * nf), jnp.float32),
        scratch_shapes=[
            pltpu.VMEM((h, w, 2 * nf), jnp.float32),
            pltpu.SemaphoreType.DMA,
        ],
    )(col_embed, row_embed)
    return jnp.transpose(q, (0, 3, 1, 2))
